# baseline (speedup 1.0000x reference)
import jax
import jax.numpy as jnp
from jax.experimental import pallas as pl


def _add_body(tok_ref, pos_ref, out_ref):
    out_ref[...] = tok_ref[...] + pos_ref[...][None, :, :]


def kernel(token_embedding, src_mask, pos_embedding):
    B, S, E = token_embedding.shape
    BS = 1024
    grid = (B, S // BS)
    return pl.pallas_call(
        _add_body,
        grid=grid,
        in_specs=[
            pl.BlockSpec((1, BS, E), lambda b, s: (b, s, 0)),
            pl.BlockSpec((BS, E), lambda b, s: (s, 0)),
        ],
        out_specs=pl.BlockSpec((1, BS, E), lambda b, s: (b, s, 0)),
        out_shape=jax.ShapeDtypeStruct((B, S, E), token_embedding.dtype),
    )(token_embedding, pos_embedding[:S])


# speedup vs baseline: 1.0501x; 1.0501x over previous
import jax
import jax.numpy as jnp
from jax.experimental import pallas as pl


def _add_body(tok_ref, pos_ref, out_ref):
    out_ref[...] = tok_ref[...] + pos_ref[...][None, :, :]


def kernel(token_embedding, src_mask, pos_embedding):
    B, S, E = token_embedding.shape
    BS = 512
    grid = (S // BS, B)
    return pl.pallas_call(
        _add_body,
        grid=grid,
        in_specs=[
            pl.BlockSpec((1, BS, E), lambda s, b: (b, s, 0)),
            pl.BlockSpec((BS, E), lambda s, b: (s, 0)),
        ],
        out_specs=pl.BlockSpec((1, BS, E), lambda s, b: (b, s, 0)),
        out_shape=jax.ShapeDtypeStruct((B, S, E), token_embedding.dtype),
    )(token_embedding, pos_embedding[:S])
